# baseline (speedup 1.0000x reference)
import functools

import jax
import jax.numpy as jnp
from jax import lax
from jax.experimental import pallas as pl
from jax.experimental.pallas import tpu as pltpu
from jax.experimental.pallas import tpu_sc as plsc

N = 10000
E = 160000
NP = 10016
EP = 163840
NWORK = 32
EPW = EP // NWORK
CH = 128
NCHUNK = EPW // CH
ROWS_PER_SUB = NP // 16
CHR = ROWS_PER_SUB // 2
BE = 2048
GRID_E = EP // BE

_mesh = plsc.VectorSubcoreMesh(core_axis_name="c", subcore_axis_name="s")


NBUF = 4


@functools.partial(
    pl.kernel, mesh=_mesh,
    out_type=jax.ShapeDtypeStruct((EP, 32), jnp.float32),
    compiler_params=pltpu.CompilerParams(use_tc_tiling_on_sc=False),
    scratch_types=[
        pltpu.VMEM((NCHUNK, CH), jnp.int32),
        pltpu.VMEM((NBUF * CH, 32), jnp.float32),
        pltpu.SemaphoreType.DMA((NBUF,)),
    ],
)
def _sc_gather(table_hbm, src2d_hbm, out_hbm, idx_v, rows_v, gsem):
    c = lax.axis_index("c")
    s = lax.axis_index("s")
    wid = c * 16 + s
    pltpu.sync_copy(src2d_hbm.at[pl.ds(wid * NCHUNK, NCHUNK)], idx_v)
    base = wid * EPW
    for b in range(NBUF):
        pltpu.async_copy(table_hbm.at[idx_v.at[b]],
                         rows_v.at[pl.ds(b * CH, CH)], gsem.at[b])

    def body(g, carry):
        for b in range(NBUF):
            j = g * NBUF + b
            buf = rows_v.at[pl.ds(b * CH, CH)]
            pltpu.make_async_copy(table_hbm.at[idx_v.at[j]], buf,
                                  gsem.at[b]).wait()
            pltpu.sync_copy(buf, out_hbm.at[pl.ds(base + j * CH, CH)])
            jn = j + NBUF

            @pl.when(jn < NCHUNK)
            def _():
                pltpu.async_copy(table_hbm.at[idx_v.at[jn]], buf, gsem.at[b])
        return carry

    lax.fori_loop(0, NCHUNK // NBUF, body, 0)


@functools.partial(
    pl.kernel, mesh=_mesh,
    out_type=jax.ShapeDtypeStruct((2, NP, 32), jnp.float32),
    compiler_params=pltpu.CompilerParams(use_tc_tiling_on_sc=False),
    scratch_types=[
        pltpu.VMEM((NCHUNK, CH), jnp.int32),
        pltpu.VMEM((NBUF * CH, 32), jnp.float32),
        pltpu.SemaphoreType.DMA((NBUF,)),
        pltpu.SemaphoreType.DMA((NBUF,)),
        pltpu.VMEM_SHARED((NP, 32), jnp.float32),
    ],
)
def _sc_scatter(msg_hbm, dst2d_hbm, zeros_hbm, out_hbm, idx_v, rows_v, lsem,
                ssem, acc_sh):
    c = lax.axis_index("c")
    s = lax.axis_index("s")
    wid = c * 16 + s
    pltpu.sync_copy(zeros_hbm.at[pl.ds(s * ROWS_PER_SUB, ROWS_PER_SUB)],
                    acc_sh.at[pl.ds(s * ROWS_PER_SUB, ROWS_PER_SUB)])
    pltpu.sync_copy(dst2d_hbm.at[pl.ds(wid * NCHUNK, NCHUNK)], idx_v)
    plsc.subcore_barrier()
    base = wid * EPW
    for b in range(NBUF):
        pltpu.async_copy(msg_hbm.at[pl.ds(base + b * CH, CH)],
                         rows_v.at[pl.ds(b * CH, CH)], lsem.at[b])

    def body(g, carry):
        for b in range(NBUF):
            j = g * NBUF + b
            buf = rows_v.at[pl.ds(b * CH, CH)]
            pltpu.make_async_copy(msg_hbm.at[pl.ds(base + j * CH, CH)], buf,
                                  lsem.at[b]).wait()
            pltpu.async_copy(buf, acc_sh.at[idx_v.at[j]], ssem.at[b],
                             add=True)
            jn = j + NBUF

            @pl.when(jn < NCHUNK)
            def _():
                pltpu.make_async_copy(buf, acc_sh.at[idx_v.at[j]],
                                      ssem.at[b]).wait()
                pltpu.async_copy(msg_hbm.at[pl.ds(base + jn * CH, CH)], buf,
                                 lsem.at[b])
        return carry

    lax.fori_loop(0, NCHUNK // NBUF, body, 0)
    for b in range(NBUF):
        pltpu.make_async_copy(rows_v.at[pl.ds(b * CH, CH)],
                              acc_sh.at[idx_v.at[NCHUNK - NBUF + b]],
                              ssem.at[b]).wait()
    plsc.subcore_barrier()
    pltpu.sync_copy(acc_sh.at[pl.ds(s * ROWS_PER_SUB, ROWS_PER_SUB)],
                    out_hbm.at[c].at[pl.ds(s * ROWS_PER_SUB, ROWS_PER_SUB)])


@functools.partial(
    pl.kernel, mesh=_mesh,
    out_type=jax.ShapeDtypeStruct((2, NP, 32), jnp.float32),
    compiler_params=pltpu.CompilerParams(use_tc_tiling_on_sc=False),
    scratch_types=[
        pltpu.VMEM((NCHUNK, CH), jnp.int32),
        pltpu.VMEM((CH, 32), jnp.float32),
        pltpu.SemaphoreType.DMA((NBUF,)),
        pltpu.VMEM_SHARED((NP, 32), jnp.float32),
    ],
)
def _sc_count(ones_hbm, dst2d_hbm, zeros_hbm, out_hbm, idx_v, rows_v, csem,
              acc_sh):
    c = lax.axis_index("c")
    s = lax.axis_index("s")
    wid = c * 16 + s
    pltpu.sync_copy(zeros_hbm.at[pl.ds(s * ROWS_PER_SUB, ROWS_PER_SUB)],
                    acc_sh.at[pl.ds(s * ROWS_PER_SUB, ROWS_PER_SUB)])
    pltpu.sync_copy(dst2d_hbm.at[pl.ds(wid * NCHUNK, NCHUNK)], idx_v)
    pltpu.sync_copy(ones_hbm, rows_v)
    plsc.subcore_barrier()
    for b in range(NBUF):
        pltpu.async_copy(rows_v, acc_sh.at[idx_v.at[b]], csem.at[b], add=True)

    def body(g, carry):
        for b in range(NBUF):
            j = g * NBUF + b
            pltpu.make_async_copy(rows_v, acc_sh.at[idx_v.at[j]],
                                  csem.at[b]).wait()
            pltpu.async_copy(rows_v, acc_sh.at[idx_v.at[j + NBUF]],
                             csem.at[b], add=True)
        return carry

    lax.fori_loop(0, NCHUNK // NBUF - 1, body, 0)
    for b in range(NBUF):
        pltpu.make_async_copy(rows_v, acc_sh.at[idx_v.at[NCHUNK - NBUF + b]],
                              csem.at[b]).wait()
    plsc.subcore_barrier()
    pltpu.sync_copy(acc_sh.at[pl.ds(s * ROWS_PER_SUB, ROWS_PER_SUB)],
                    out_hbm.at[c].at[pl.ds(s * ROWS_PER_SUB, ROWS_PER_SUB)])


def _make_gfin(with_xt):
    outs = [
        jax.ShapeDtypeStruct((EP, 32), jnp.float32),
        jax.ShapeDtypeStruct((NP, 32), jnp.float32),
    ]
    if with_xt:
        outs.append(jax.ShapeDtypeStruct((NP, 32), jnp.float32))

    def body(p_hbm, invc_hbm, r_hbm, xt_hbm, src2d_hbm, *refs):
        if with_xt:
            out_hs, out_tab, out_h26 = refs[0], refs[1], refs[2]
            scr = refs[3:]
        else:
            out_hs, out_tab = refs[0], refs[1]
            out_h26 = None
            scr = refs[2:]
        idx_v, rows_v, gsem, pa_v, pb_v, iv_v, rv_v, xv_v, tab_sh = scr
        c = lax.axis_index("c")
        s = lax.axis_index("s")
        wid = c * 16 + s
        pltpu.sync_copy(src2d_hbm.at[pl.ds(wid * NCHUNK, NCHUNK)], idx_v)
        for k in range(2):
            rs = s * ROWS_PER_SUB + k * CHR
            pltpu.sync_copy(p_hbm.at[0].at[pl.ds(rs, CHR)], pa_v)
            pltpu.sync_copy(p_hbm.at[1].at[pl.ds(rs, CHR)], pb_v)
            pltpu.sync_copy(invc_hbm.at[pl.ds(rs, CHR)], iv_v)
            pltpu.sync_copy(r_hbm.at[pl.ds(rs, CHR)], rv_v)
            if with_xt:
                pltpu.sync_copy(xt_hbm.at[pl.ds(rs, CHR)], xv_v)

            def finrow(i, carry):
                for half in range(2):
                    sl = pl.ds(16 * half, 16)
                    v = (pa_v[i, sl] + pb_v[i, sl]) * iv_v[i, sl] + rv_v[i, sl]
                    h26 = jnp.maximum(v, 0.0)
                    if with_xt:
                        pa_v[i, sl] = h26
                        pb_v[i, sl] = h26 + xv_v[i, sl]
                    else:
                        pb_v[i, sl] = h26
                return carry

            lax.fori_loop(0, CHR, finrow, 0)
            if with_xt:
                pltpu.sync_copy(pa_v, out_h26.at[pl.ds(rs, CHR)])
            pltpu.sync_copy(pb_v, out_tab.at[pl.ds(rs, CHR)])
            pltpu.sync_copy(pb_v, tab_sh.at[pl.ds(rs, CHR)])
        plsc.subcore_barrier()

        base = wid * EPW
        for b in range(NBUF):
            pltpu.async_copy(tab_sh.at[idx_v.at[b]],
                             rows_v.at[pl.ds(b * CH, CH)], gsem.at[b])

        def gbody(g, carry):
            for b in range(NBUF):
                j = g * NBUF + b
                buf = rows_v.at[pl.ds(b * CH, CH)]
                pltpu.make_async_copy(tab_sh.at[idx_v.at[j]], buf,
                                      gsem.at[b]).wait()
                pltpu.sync_copy(buf, out_hs.at[pl.ds(base + j * CH, CH)])
                jn = j + NBUF

                @pl.when(jn < NCHUNK)
                def _():
                    pltpu.async_copy(tab_sh.at[idx_v.at[jn]], buf, gsem.at[b])
            return carry

        lax.fori_loop(0, NCHUNK // NBUF, gbody, 0)

    return functools.partial(
        pl.kernel, mesh=_mesh,
        out_type=outs,
        compiler_params=pltpu.CompilerParams(use_tc_tiling_on_sc=False),
        scratch_types=[
            pltpu.VMEM((NCHUNK, CH), jnp.int32),
            pltpu.VMEM((NBUF * CH, 32), jnp.float32),
            pltpu.SemaphoreType.DMA((NBUF,)),
            pltpu.VMEM((CHR, 32), jnp.float32),
            pltpu.VMEM((CHR, 32), jnp.float32),
            pltpu.VMEM((CHR, 32), jnp.float32),
            pltpu.VMEM((CHR, 32), jnp.float32),
            pltpu.VMEM((CHR, 32), jnp.float32),
            pltpu.VMEM_SHARED((NP, 32), jnp.float32),
        ],
    )(body)


_sc_gfin_mid = _make_gfin(False)
_sc_gfin_step = _make_gfin(True)

NCHUNK2 = EP // 16 // CH


def _make_scatfin(with_xt):
    outs = [
        jax.ShapeDtypeStruct((EP, 32), jnp.float32),
        jax.ShapeDtypeStruct((NP, 32), jnp.float32),
    ]
    if with_xt:
        outs.append(jax.ShapeDtypeStruct((NP, 32), jnp.float32))

    def body(msg_hbm, dst2d_hbm, src2d_hbm, zeros_hbm, invc_hbm, r_hbm,
             xt_hbm, *refs):
        if with_xt:
            out_hs, out_tab, out_h26 = refs[0], refs[1], refs[2]
            scr = refs[3:]
        else:
            out_hs, out_tab = refs[0], refs[1]
            out_h26 = None
            scr = refs[2:]
        (didx_v, gidx_v, rows_v, lsem, ssem,
         pa_v, iv_v, rv_v, xv_v, acc_sh) = scr
        c = lax.axis_index("c")
        s = lax.axis_index("s")
        wid = c * 16 + s
        pltpu.sync_copy(zeros_hbm.at[pl.ds(s * ROWS_PER_SUB, ROWS_PER_SUB)],
                        acc_sh.at[pl.ds(s * ROWS_PER_SUB, ROWS_PER_SUB)])
        pltpu.sync_copy(dst2d_hbm.at[pl.ds(s * NCHUNK2, NCHUNK2)], didx_v)
        pltpu.sync_copy(src2d_hbm.at[pl.ds(wid * NCHUNK, NCHUNK)], gidx_v)
        plsc.subcore_barrier()

        sbase = s * NCHUNK2 * CH
        for b in range(NBUF):
            pltpu.async_copy(msg_hbm.at[pl.ds(sbase + b * CH, CH)],
                             rows_v.at[pl.ds(b * CH, CH)], lsem.at[b])

        def sbody(g, carry):
            for b in range(NBUF):
                j = g * NBUF + b
                buf = rows_v.at[pl.ds(b * CH, CH)]
                pltpu.make_async_copy(msg_hbm.at[pl.ds(sbase + j * CH, CH)],
                                      buf, lsem.at[b]).wait()
                pltpu.async_copy(buf, acc_sh.at[didx_v.at[j]], ssem.at[b],
                                 add=True)
                jn = j + NBUF

                @pl.when(jn < NCHUNK2)
                def _():
                    pltpu.make_async_copy(buf, acc_sh.at[didx_v.at[j]],
                                          ssem.at[b]).wait()
                    pltpu.async_copy(msg_hbm.at[pl.ds(sbase + jn * CH, CH)],
                                     buf, lsem.at[b])
            return carry

        lax.fori_loop(0, NCHUNK2 // NBUF, sbody, 0)
        for b in range(NBUF):
            pltpu.make_async_copy(
                rows_v.at[pl.ds(b * CH, CH)],
                acc_sh.at[didx_v.at[NCHUNK2 - NBUF + b]], ssem.at[b]).wait()
        plsc.subcore_barrier()

        for k in range(2):
            rs = s * ROWS_PER_SUB + k * CHR
            pltpu.sync_copy(acc_sh.at[pl.ds(rs, CHR)], pa_v)
            pltpu.sync_copy(invc_hbm.at[pl.ds(rs, CHR)], iv_v)
            pltpu.sync_copy(r_hbm.at[pl.ds(rs, CHR)], rv_v)
            if with_xt:
                pltpu.sync_copy(xt_hbm.at[pl.ds(rs, CHR)], xv_v)

            def finrow(i, carry):
                for half in range(2):
                    sl = pl.ds(16 * half, 16)
                    h26 = jnp.maximum(
                        pa_v[i, sl] * iv_v[i, sl] + rv_v[i, sl], 0.0)
                    if with_xt:
                        pa_v[i, sl] = h26
                        rv_v[i, sl] = h26 + xv_v[i, sl]
                    else:
                        rv_v[i, sl] = h26
                return carry

            lax.fori_loop(0, CHR, finrow, 0)
            if with_xt:
                pltpu.sync_copy(pa_v, out_h26.at[pl.ds(rs, CHR)])
            pltpu.sync_copy(rv_v, out_tab.at[pl.ds(rs, CHR)])
            pltpu.sync_copy(rv_v, acc_sh.at[pl.ds(rs, CHR)])
        plsc.subcore_barrier()

        base = wid * EPW
        for b in range(NBUF):
            pltpu.async_copy(acc_sh.at[gidx_v.at[b]],
                             rows_v.at[pl.ds(b * CH, CH)], lsem.at[b])

        def gbody(g, carry):
            for b in range(NBUF):
                j = g * NBUF + b
                buf = rows_v.at[pl.ds(b * CH, CH)]
                pltpu.make_async_copy(acc_sh.at[gidx_v.at[j]], buf,
                                      lsem.at[b]).wait()
                pltpu.sync_copy(buf, out_hs.at[pl.ds(base + j * CH, CH)])
                jn = j + NBUF

                @pl.when(jn < NCHUNK)
                def _():
                    pltpu.async_copy(acc_sh.at[gidx_v.at[jn]], buf,
                                     lsem.at[b])
            return carry

        lax.fori_loop(0, NCHUNK // NBUF, gbody, 0)

    return functools.partial(
        pl.kernel, mesh=_mesh,
        out_type=outs,
        compiler_params=pltpu.CompilerParams(use_tc_tiling_on_sc=False),
        scratch_types=[
            pltpu.VMEM((NCHUNK2, CH), jnp.int32),
            pltpu.VMEM((NCHUNK, CH), jnp.int32),
            pltpu.VMEM((NBUF * CH, 32), jnp.float32),
            pltpu.SemaphoreType.DMA((NBUF,)),
            pltpu.SemaphoreType.DMA((NBUF,)),
            pltpu.VMEM((CHR, 32), jnp.float32),
            pltpu.VMEM((CHR, 32), jnp.float32),
            pltpu.VMEM((CHR, 32), jnp.float32),
            pltpu.VMEM((CHR, 32), jnp.float32),
            pltpu.VMEM_SHARED((NP, 32), jnp.float32),
        ],
    )(body)


_sc_scatfin_mid = _make_scatfin(False)
_sc_scatfin_step = _make_scatfin(True)


def _msg_body(hs_ref, attr_ref, w1_ref, b1_ref, t_ref, w2s_ref, bm_ref,
              htab_ref, root_ref, rbias_ref, cp_ref,
              out_ref, r_ref, invc_ref):
    @pl.when(pl.program_id(0) == 0)
    def _():
        r_ref[...] = (jnp.dot(htab_ref[...], root_ref[...],
                              preferred_element_type=jnp.float32)
                      + rbias_ref[...])
        cnt = cp_ref[0][:, 0:1] + cp_ref[1][:, 0:1]
        invc_ref[...] = jnp.broadcast_to(1.0 / jnp.maximum(cnt, 1.0),
                                         (NP, 32))

    hs = hs_ref[...]
    hsb = hs.astype(jnp.bfloat16)
    e = jnp.maximum(attr_ref[...] * w1_ref[...] + b1_ref[...], 0.0)
    e_tile = pltpu.repeat(e.astype(jnp.bfloat16), 32, axis=1)
    h_exp = jnp.dot(hsb, t_ref[...],
                    preferred_element_type=jnp.float32).astype(jnp.bfloat16)
    u = h_exp * e_tile
    out_ref[...] = (
        jnp.dot(u, w2s_ref[...], preferred_element_type=jnp.float32)
        + jnp.dot(hs, bm_ref[...], preferred_element_type=jnp.float32))


def _msg(hsrc, attrp, w1, b1, tmat, w2s, bmat, htab, root, rbias, cp):
    return pl.pallas_call(
        _msg_body,
        grid=(GRID_E,),
        in_specs=[
            pl.BlockSpec((BE, 32), lambda j: (j, 0)),
            pl.BlockSpec((BE, 1), lambda j: (j, 0)),
            pl.BlockSpec((1, 16), lambda j: (0, 0)),
            pl.BlockSpec((1, 16), lambda j: (0, 0)),
            pl.BlockSpec((32, 512), lambda j: (0, 0)),
            pl.BlockSpec((512, 32), lambda j: (0, 0)),
            pl.BlockSpec((32, 32), lambda j: (0, 0)),
            pl.BlockSpec((NP, 32), lambda j: (0, 0)),
            pl.BlockSpec((32, 32), lambda j: (0, 0)),
            pl.BlockSpec((1, 32), lambda j: (0, 0)),
            pl.BlockSpec((2, NP, 32), lambda j: (0, 0, 0)),
        ],
        out_specs=[
            pl.BlockSpec((BE, 32), lambda j: (j, 0)),
            pl.BlockSpec((NP, 32), lambda j: (0, 0)),
            pl.BlockSpec((NP, 32), lambda j: (0, 0)),
        ],
        out_shape=[
            jax.ShapeDtypeStruct((EP, 32), jnp.float32),
            jax.ShapeDtypeStruct((NP, 32), jnp.float32),
            jax.ShapeDtypeStruct((NP, 32), jnp.float32),
        ],
    )(hsrc, attrp, w1, b1, tmat, w2s, bmat, htab, root, rbias, cp)


def _prologue_body(bnd_ref, w1_ref, b1_ref, w2_ref, b2_ref, xt_ref, out_ref):
    h0 = jnp.maximum(bnd_ref[...] * w1_ref[...] + b1_ref[...], 0.0)
    h26 = jnp.maximum(
        jnp.dot(h0, w2_ref[...], preferred_element_type=jnp.float32)
        + b2_ref[...], 0.0)
    out_ref[...] = h26 + xt_ref[...]


def _prologue(bnd, fc1_w, fc1_b, fc2p, fc2bp, xt1):
    return pl.pallas_call(
        _prologue_body,
        out_shape=jax.ShapeDtypeStruct((NP, 32), jnp.float32),
    )(bnd, fc1_w, fc1_b, fc2p, fc2bp, xt1)


def _epi_body(h26a_ref, h26b_ref, p_ref, invc_ref, r_ref, fc3_ref, fc3b_ref,
              fc4_ref, fc4b_ref, y1_ref, y2_ref, y3_ref):
    h26c = jnp.maximum(
        (p_ref[0] + p_ref[1]) * invc_ref[...] + r_ref[...], 0.0)

    def head(h26, y_ref):
        z = jnp.maximum(
            jnp.dot(h26, fc3_ref[...], preferred_element_type=jnp.float32)
            + fc3b_ref[...], 0.0)
        y_ref[...] = (jnp.dot(z, fc4_ref[...],
                              preferred_element_type=jnp.float32)
                      + fc4b_ref[...])

    head(h26a_ref[...], y1_ref)
    head(h26b_ref[...], y2_ref)
    head(h26c, y3_ref)


def _epilogue(h26a, h26b, p, invc, r, fc3p, fc3b, fc4, fc4b):
    return pl.pallas_call(
        _epi_body,
        out_shape=[
            jax.ShapeDtypeStruct((NP, 1), jnp.float32),
            jax.ShapeDtypeStruct((NP, 1), jnp.float32),
            jax.ShapeDtypeStruct((NP, 1), jnp.float32),
        ],
    )(h26a, h26b, p, invc, r, fc3p, fc3b, fc4, fc4b)


def kernel(x, t, edge_index, edge_attr, y, fc1_w, fc1_b, fc2_w, fc2_b,
           nn1_w1, nn1_b1, nn1_w2, nn1_b2, conv1_root, conv1_bias,
           nn3_w1, nn3_b1, nn3_w2, nn3_b2, conv3_root, conv3_bias,
           fc3_w, fc3_b, fc4_w, fc4_b):
    f32 = jnp.float32
    pad_e = EP - E

    src = edge_index[0].astype(jnp.int32)
    dst = edge_index[1].astype(jnp.int32)
    src2d = jnp.concatenate([src, jnp.zeros((pad_e,), jnp.int32)]).reshape(-1, CH)
    dst2d = jnp.concatenate(
        [dst, jnp.full((pad_e,), N, jnp.int32)]).reshape(-1, CH)
    attrp = jnp.pad(edge_attr, ((0, pad_e), (0, 0)))

    w2s1 = nn1_w2.reshape(16, 32, 32).transpose(1, 0, 2).reshape(512, 32)
    bm1 = nn1_b2.reshape(32, 32)
    w2s3 = jnp.pad(nn3_w2.reshape(16, 32, 26),
                   ((0, 0), (0, 0), (0, 6))).transpose(1, 0, 2).reshape(512, 32)
    bm3 = jnp.pad(nn3_b2.reshape(32, 26), ((0, 0), (0, 6)))
    tmat = jnp.kron(jnp.eye(32, dtype=jnp.bfloat16),
                    jnp.ones((1, 16), jnp.bfloat16))
    w2s1 = w2s1.astype(jnp.bfloat16)
    w2s3 = w2s3.astype(jnp.bfloat16)
    w1e1 = nn1_w1
    b1e1 = nn1_b1.reshape(1, 16)
    w1e3 = nn3_w1
    b1e3 = nn3_b1.reshape(1, 16)
    root1 = conv1_root
    bias1 = conv1_bias.reshape(1, 32)
    root3 = jnp.pad(conv3_root, ((0, 0), (0, 6)))
    bias3 = jnp.pad(conv3_bias, (0, 6)).reshape(1, 32)
    fc2p = jnp.pad(fc2_w, ((0, 0), (0, 6)))
    fc2bp = jnp.pad(fc2_b, (0, 6)).reshape(1, 32)
    fc3p = jnp.pad(fc3_w, ((0, 6), (0, 0)))
    fc3b = fc3_b.reshape(1, 32)
    fc4b = fc4_b.reshape(1, 1)

    xp = jnp.pad(x, ((0, NP - N), (0, 0)))
    zeros26 = jnp.zeros((NP, 26), f32)

    def xt_for(ti):
        return jnp.concatenate(
            [zeros26, xp, xp, xp, jnp.broadcast_to(ti, (NP, 3))], axis=1)

    zeros_np = jnp.zeros((NP, 32), f32)
    ones_ch = jnp.ones((CH, 32), f32)

    cp = _sc_count(ones_ch, dst2d, zeros_np)

    bnd = jnp.pad(y[0].reshape(-1, 1), ((0, NP - N), (0, 0)))
    h32 = _prologue(bnd, fc1_w, fc1_b.reshape(1, 32), fc2p, fc2bp, xt_for(t[1]))

    h26s = []
    T = t.shape[0]
    hs1 = _sc_gather(h32, src2d)
    for i in range(1, T):
        m1, r1, invc = _msg(hs1, attrp, w1e1, b1e1, tmat, w2s1, bm1,
                            h32, root1, bias1, cp)
        hs3, hl1 = _sc_scatfin_mid(m1, dst2d, src2d, zeros_np, invc, r1,
                                   invc)
        m3, r3, invc = _msg(hs3, attrp, w1e3, b1e3, tmat, w2s3, bm3,
                            hl1, root3, bias3, cp)
        if i + 1 < T:
            hs1, h32, h26 = _sc_scatfin_step(m3, dst2d, src2d, zeros_np,
                                             invc, r3, xt_for(t[i + 1]))
            h26s.append(h26)
        else:
            p3 = _sc_scatter(m3, dst2d, zeros_np)

    y1, y2, y3 = _epilogue(h26s[0], h26s[1], p3, invc, r3,
                           fc3p, fc3b, fc4_w, fc4b)
    return jnp.concatenate([y[0], y1[:N, 0], y2[:N, 0], y3[:N, 0]])


# speedup vs baseline: 1.0160x; 1.0160x over previous
import functools

import jax
import jax.numpy as jnp
from jax import lax
from jax.experimental import pallas as pl
from jax.experimental.pallas import tpu as pltpu
from jax.experimental.pallas import tpu_sc as plsc

N = 10000
E = 160000
NP = 10016
EP = 163840
NWORK = 32
EPW = EP // NWORK
CH = 128
NCHUNK = EPW // CH
ROWS_PER_SUB = NP // 16
CHR = ROWS_PER_SUB // 2
BE = 2048
GRID_E = EP // BE

_mesh = plsc.VectorSubcoreMesh(core_axis_name="c", subcore_axis_name="s")


NBUF = 4


@functools.partial(
    pl.kernel, mesh=_mesh,
    out_type=jax.ShapeDtypeStruct((EP, 32), jnp.float32),
    compiler_params=pltpu.CompilerParams(use_tc_tiling_on_sc=False),
    scratch_types=[
        pltpu.VMEM((NCHUNK, CH), jnp.int32),
        pltpu.VMEM((NBUF * CH, 32), jnp.float32),
        pltpu.SemaphoreType.DMA((NBUF,)),
    ],
)
def _sc_gather(table_hbm, src2d_hbm, out_hbm, idx_v, rows_v, gsem):
    c = lax.axis_index("c")
    s = lax.axis_index("s")
    wid = c * 16 + s
    pltpu.sync_copy(src2d_hbm.at[pl.ds(wid * NCHUNK, NCHUNK)], idx_v)
    base = wid * EPW
    for b in range(NBUF):
        pltpu.async_copy(table_hbm.at[idx_v.at[b]],
                         rows_v.at[pl.ds(b * CH, CH)], gsem.at[b])

    def body(g, carry):
        for b in range(NBUF):
            j = g * NBUF + b
            buf = rows_v.at[pl.ds(b * CH, CH)]
            pltpu.make_async_copy(table_hbm.at[idx_v.at[j]], buf,
                                  gsem.at[b]).wait()
            pltpu.sync_copy(buf, out_hbm.at[pl.ds(base + j * CH, CH)])
            jn = j + NBUF

            @pl.when(jn < NCHUNK)
            def _():
                pltpu.async_copy(table_hbm.at[idx_v.at[jn]], buf, gsem.at[b])
        return carry

    lax.fori_loop(0, NCHUNK // NBUF, body, 0)


@functools.partial(
    pl.kernel, mesh=_mesh,
    out_type=jax.ShapeDtypeStruct((2, NP, 32), jnp.float32),
    compiler_params=pltpu.CompilerParams(use_tc_tiling_on_sc=False),
    scratch_types=[
        pltpu.VMEM((NCHUNK, CH), jnp.int32),
        pltpu.VMEM((NBUF * CH, 32), jnp.float32),
        pltpu.SemaphoreType.DMA((NBUF,)),
        pltpu.SemaphoreType.DMA((NBUF,)),
        pltpu.VMEM_SHARED((NP, 32), jnp.float32),
    ],
)
def _sc_scatter(msg_hbm, dst2d_hbm, zeros_hbm, out_hbm, idx_v, rows_v, lsem,
                ssem, acc_sh):
    c = lax.axis_index("c")
    s = lax.axis_index("s")
    wid = c * 16 + s
    pltpu.sync_copy(zeros_hbm.at[pl.ds(s * ROWS_PER_SUB, ROWS_PER_SUB)],
                    acc_sh.at[pl.ds(s * ROWS_PER_SUB, ROWS_PER_SUB)])
    pltpu.sync_copy(dst2d_hbm.at[pl.ds(wid * NCHUNK, NCHUNK)], idx_v)
    plsc.subcore_barrier()
    base = wid * EPW
    for b in range(NBUF):
        pltpu.async_copy(msg_hbm.at[pl.ds(base + b * CH, CH)],
                         rows_v.at[pl.ds(b * CH, CH)], lsem.at[b])

    def body(g, carry):
        for b in range(NBUF):
            j = g * NBUF + b
            buf = rows_v.at[pl.ds(b * CH, CH)]
            pltpu.make_async_copy(msg_hbm.at[pl.ds(base + j * CH, CH)], buf,
                                  lsem.at[b]).wait()
            pltpu.async_copy(buf, acc_sh.at[idx_v.at[j]], ssem.at[b],
                             add=True)
            jn = j + NBUF

            @pl.when(jn < NCHUNK)
            def _():
                pltpu.make_async_copy(buf, acc_sh.at[idx_v.at[j]],
                                      ssem.at[b]).wait()
                pltpu.async_copy(msg_hbm.at[pl.ds(base + jn * CH, CH)], buf,
                                 lsem.at[b])
        return carry

    lax.fori_loop(0, NCHUNK // NBUF, body, 0)
    for b in range(NBUF):
        pltpu.make_async_copy(rows_v.at[pl.ds(b * CH, CH)],
                              acc_sh.at[idx_v.at[NCHUNK - NBUF + b]],
                              ssem.at[b]).wait()
    plsc.subcore_barrier()
    pltpu.sync_copy(acc_sh.at[pl.ds(s * ROWS_PER_SUB, ROWS_PER_SUB)],
                    out_hbm.at[c].at[pl.ds(s * ROWS_PER_SUB, ROWS_PER_SUB)])


@functools.partial(
    pl.kernel, mesh=_mesh,
    out_type=jax.ShapeDtypeStruct((2, NP, 32), jnp.float32),
    compiler_params=pltpu.CompilerParams(use_tc_tiling_on_sc=False),
    scratch_types=[
        pltpu.VMEM((NCHUNK, CH), jnp.int32),
        pltpu.VMEM((CH, 32), jnp.float32),
        pltpu.SemaphoreType.DMA((NBUF,)),
        pltpu.VMEM_SHARED((NP, 32), jnp.float32),
    ],
)
def _sc_count(ones_hbm, dst2d_hbm, zeros_hbm, out_hbm, idx_v, rows_v, csem,
              acc_sh):
    c = lax.axis_index("c")
    s = lax.axis_index("s")
    wid = c * 16 + s
    pltpu.sync_copy(zeros_hbm.at[pl.ds(s * ROWS_PER_SUB, ROWS_PER_SUB)],
                    acc_sh.at[pl.ds(s * ROWS_PER_SUB, ROWS_PER_SUB)])
    pltpu.sync_copy(dst2d_hbm.at[pl.ds(wid * NCHUNK, NCHUNK)], idx_v)
    pltpu.sync_copy(ones_hbm, rows_v)
    plsc.subcore_barrier()
    for b in range(NBUF):
        pltpu.async_copy(rows_v, acc_sh.at[idx_v.at[b]], csem.at[b], add=True)

    def body(g, carry):
        for b in range(NBUF):
            j = g * NBUF + b
            pltpu.make_async_copy(rows_v, acc_sh.at[idx_v.at[j]],
                                  csem.at[b]).wait()
            pltpu.async_copy(rows_v, acc_sh.at[idx_v.at[j + NBUF]],
                             csem.at[b], add=True)
        return carry

    lax.fori_loop(0, NCHUNK // NBUF - 1, body, 0)
    for b in range(NBUF):
        pltpu.make_async_copy(rows_v, acc_sh.at[idx_v.at[NCHUNK - NBUF + b]],
                              csem.at[b]).wait()
    plsc.subcore_barrier()
    pltpu.sync_copy(acc_sh.at[pl.ds(s * ROWS_PER_SUB, ROWS_PER_SUB)],
                    out_hbm.at[c].at[pl.ds(s * ROWS_PER_SUB, ROWS_PER_SUB)])


def _make_gfin(with_xt):
    outs = [
        jax.ShapeDtypeStruct((EP, 32), jnp.float32),
        jax.ShapeDtypeStruct((NP, 32), jnp.float32),
    ]
    if with_xt:
        outs.append(jax.ShapeDtypeStruct((NP, 32), jnp.float32))

    def body(p_hbm, invc_hbm, r_hbm, xt_hbm, src2d_hbm, *refs):
        if with_xt:
            out_hs, out_tab, out_h26 = refs[0], refs[1], refs[2]
            scr = refs[3:]
        else:
            out_hs, out_tab = refs[0], refs[1]
            out_h26 = None
            scr = refs[2:]
        idx_v, rows_v, gsem, pa_v, pb_v, iv_v, rv_v, xv_v, tab_sh = scr
        c = lax.axis_index("c")
        s = lax.axis_index("s")
        wid = c * 16 + s
        pltpu.sync_copy(src2d_hbm.at[pl.ds(wid * NCHUNK, NCHUNK)], idx_v)
        for k in range(2):
            rs = s * ROWS_PER_SUB + k * CHR
            pltpu.sync_copy(p_hbm.at[0].at[pl.ds(rs, CHR)], pa_v)
            pltpu.sync_copy(p_hbm.at[1].at[pl.ds(rs, CHR)], pb_v)
            pltpu.sync_copy(invc_hbm.at[pl.ds(rs, CHR)], iv_v)
            pltpu.sync_copy(r_hbm.at[pl.ds(rs, CHR)], rv_v)
            if with_xt:
                pltpu.sync_copy(xt_hbm.at[pl.ds(rs, CHR)], xv_v)

            def finrow(i, carry):
                for half in range(2):
                    sl = pl.ds(16 * half, 16)
                    v = (pa_v[i, sl] + pb_v[i, sl]) * iv_v[i, sl] + rv_v[i, sl]
                    h26 = jnp.maximum(v, 0.0)
                    if with_xt:
                        pa_v[i, sl] = h26
                        pb_v[i, sl] = h26 + xv_v[i, sl]
                    else:
                        pb_v[i, sl] = h26
                return carry

            lax.fori_loop(0, CHR, finrow, 0)
            if with_xt:
                pltpu.sync_copy(pa_v, out_h26.at[pl.ds(rs, CHR)])
            pltpu.sync_copy(pb_v, out_tab.at[pl.ds(rs, CHR)])
            pltpu.sync_copy(pb_v, tab_sh.at[pl.ds(rs, CHR)])
        plsc.subcore_barrier()

        base = wid * EPW
        for b in range(NBUF):
            pltpu.async_copy(tab_sh.at[idx_v.at[b]],
                             rows_v.at[pl.ds(b * CH, CH)], gsem.at[b])

        def gbody(g, carry):
            for b in range(NBUF):
                j = g * NBUF + b
                buf = rows_v.at[pl.ds(b * CH, CH)]
                pltpu.make_async_copy(tab_sh.at[idx_v.at[j]], buf,
                                      gsem.at[b]).wait()
                pltpu.sync_copy(buf, out_hs.at[pl.ds(base + j * CH, CH)])
                jn = j + NBUF

                @pl.when(jn < NCHUNK)
                def _():
                    pltpu.async_copy(tab_sh.at[idx_v.at[jn]], buf, gsem.at[b])
            return carry

        lax.fori_loop(0, NCHUNK // NBUF, gbody, 0)

    return functools.partial(
        pl.kernel, mesh=_mesh,
        out_type=outs,
        compiler_params=pltpu.CompilerParams(use_tc_tiling_on_sc=False),
        scratch_types=[
            pltpu.VMEM((NCHUNK, CH), jnp.int32),
            pltpu.VMEM((NBUF * CH, 32), jnp.float32),
            pltpu.SemaphoreType.DMA((NBUF,)),
            pltpu.VMEM((CHR, 32), jnp.float32),
            pltpu.VMEM((CHR, 32), jnp.float32),
            pltpu.VMEM((CHR, 32), jnp.float32),
            pltpu.VMEM((CHR, 32), jnp.float32),
            pltpu.VMEM((CHR, 32), jnp.float32),
            pltpu.VMEM_SHARED((NP, 32), jnp.float32),
        ],
    )(body)


_sc_gfin_mid = _make_gfin(False)
_sc_gfin_step = _make_gfin(True)


@functools.partial(
    pl.kernel, mesh=_mesh,
    out_type=[
        jax.ShapeDtypeStruct((EP, 32), jnp.float32),
        jax.ShapeDtypeStruct((2, NP, 32), jnp.float32),
    ],
    compiler_params=pltpu.CompilerParams(use_tc_tiling_on_sc=False),
    scratch_types=[
        pltpu.VMEM((NCHUNK, CH), jnp.int32),
        pltpu.VMEM((NCHUNK, CH), jnp.int32),
        pltpu.VMEM((NBUF * CH, 32), jnp.float32),
        pltpu.VMEM((CH, 32), jnp.float32),
        pltpu.SemaphoreType.DMA((NBUF,)),
        pltpu.SemaphoreType.DMA((NBUF,)),
        pltpu.VMEM_SHARED((NP, 32), jnp.float32),
    ],
)
def _sc_count_gather(table_hbm, src2d_hbm, dst2d_hbm, ones_hbm, zeros_hbm,
                     out_hs, out_cp, gidx_v, didx_v, rows_v, ones_v, gsem,
                     csem, acc_sh):
    c = lax.axis_index("c")
    s = lax.axis_index("s")
    wid = c * 16 + s
    pltpu.sync_copy(zeros_hbm.at[pl.ds(s * ROWS_PER_SUB, ROWS_PER_SUB)],
                    acc_sh.at[pl.ds(s * ROWS_PER_SUB, ROWS_PER_SUB)])
    pltpu.sync_copy(src2d_hbm.at[pl.ds(wid * NCHUNK, NCHUNK)], gidx_v)
    pltpu.sync_copy(dst2d_hbm.at[pl.ds(wid * NCHUNK, NCHUNK)], didx_v)
    pltpu.sync_copy(ones_hbm, ones_v)
    plsc.subcore_barrier()
    base = wid * EPW
    for b in range(NBUF):
        pltpu.async_copy(ones_v, acc_sh.at[didx_v.at[b]], csem.at[b],
                         add=True)
        pltpu.async_copy(table_hbm.at[gidx_v.at[b]],
                         rows_v.at[pl.ds(b * CH, CH)], gsem.at[b])

    def body(g, carry):
        for b in range(NBUF):
            j = g * NBUF + b
            pltpu.make_async_copy(ones_v, acc_sh.at[didx_v.at[j]],
                                  csem.at[b]).wait()

            @pl.when(j + NBUF < NCHUNK)
            def _():
                pltpu.async_copy(ones_v, acc_sh.at[didx_v.at[j + NBUF]],
                                 csem.at[b], add=True)
            buf = rows_v.at[pl.ds(b * CH, CH)]
            pltpu.make_async_copy(table_hbm.at[gidx_v.at[j]], buf,
                                  gsem.at[b]).wait()
            pltpu.sync_copy(buf, out_hs.at[pl.ds(base + j * CH, CH)])

            @pl.when(j + NBUF < NCHUNK)
            def _():
                pltpu.async_copy(table_hbm.at[gidx_v.at[j + NBUF]], buf,
                                 gsem.at[b])
        return carry

    lax.fori_loop(0, NCHUNK // NBUF, body, 0)
    plsc.subcore_barrier()
    pltpu.sync_copy(acc_sh.at[pl.ds(s * ROWS_PER_SUB, ROWS_PER_SUB)],
                    out_cp.at[c].at[pl.ds(s * ROWS_PER_SUB, ROWS_PER_SUB)])

NCHUNK2 = EP // 16 // CH


def _make_scatfin(with_xt):
    outs = [
        jax.ShapeDtypeStruct((EP, 32), jnp.float32),
        jax.ShapeDtypeStruct((NP, 32), jnp.float32),
    ]
    if with_xt:
        outs.append(jax.ShapeDtypeStruct((NP, 32), jnp.float32))

    def body(msg_hbm, dst2d_hbm, src2d_hbm, zeros_hbm, invc_hbm, r_hbm,
             xt_hbm, *refs):
        if with_xt:
            out_hs, out_tab, out_h26 = refs[0], refs[1], refs[2]
            scr = refs[3:]
        else:
            out_hs, out_tab = refs[0], refs[1]
            out_h26 = None
            scr = refs[2:]
        (didx_v, gidx_v, rows_v, lsem, ssem,
         pa_v, iv_v, rv_v, xv_v, acc_sh) = scr
        c = lax.axis_index("c")
        s = lax.axis_index("s")
        wid = c * 16 + s
        pltpu.sync_copy(zeros_hbm.at[pl.ds(s * ROWS_PER_SUB, ROWS_PER_SUB)],
                        acc_sh.at[pl.ds(s * ROWS_PER_SUB, ROWS_PER_SUB)])
        pltpu.sync_copy(dst2d_hbm.at[pl.ds(s * NCHUNK2, NCHUNK2)], didx_v)
        pltpu.sync_copy(src2d_hbm.at[pl.ds(wid * NCHUNK, NCHUNK)], gidx_v)
        plsc.subcore_barrier()

        sbase = s * NCHUNK2 * CH
        for b in range(NBUF):
            pltpu.async_copy(msg_hbm.at[pl.ds(sbase + b * CH, CH)],
                             rows_v.at[pl.ds(b * CH, CH)], lsem.at[b])

        def sbody(g, carry):
            for b in range(NBUF):
                j = g * NBUF + b
                buf = rows_v.at[pl.ds(b * CH, CH)]
                pltpu.make_async_copy(msg_hbm.at[pl.ds(sbase + j * CH, CH)],
                                      buf, lsem.at[b]).wait()
                pltpu.async_copy(buf, acc_sh.at[didx_v.at[j]], ssem.at[b],
                                 add=True)
                jn = j + NBUF

                @pl.when(jn < NCHUNK2)
                def _():
                    pltpu.make_async_copy(buf, acc_sh.at[didx_v.at[j]],
                                          ssem.at[b]).wait()
                    pltpu.async_copy(msg_hbm.at[pl.ds(sbase + jn * CH, CH)],
                                     buf, lsem.at[b])
            return carry

        lax.fori_loop(0, NCHUNK2 // NBUF, sbody, 0)
        for b in range(NBUF):
            pltpu.make_async_copy(
                rows_v.at[pl.ds(b * CH, CH)],
                acc_sh.at[didx_v.at[NCHUNK2 - NBUF + b]], ssem.at[b]).wait()
        plsc.subcore_barrier()

        for k in range(2):
            rs = s * ROWS_PER_SUB + k * CHR
            pltpu.sync_copy(acc_sh.at[pl.ds(rs, CHR)], pa_v)
            pltpu.sync_copy(invc_hbm.at[pl.ds(rs, CHR)], iv_v)
            pltpu.sync_copy(r_hbm.at[pl.ds(rs, CHR)], rv_v)
            if with_xt:
                pltpu.sync_copy(xt_hbm.at[pl.ds(rs, CHR)], xv_v)

            def finrow(i, carry):
                for half in range(2):
                    sl = pl.ds(16 * half, 16)
                    h26 = jnp.maximum(
                        pa_v[i, sl] * iv_v[i, sl] + rv_v[i, sl], 0.0)
                    if with_xt:
                        pa_v[i, sl] = h26
                        rv_v[i, sl] = h26 + xv_v[i, sl]
                    else:
                        rv_v[i, sl] = h26
                return carry

            lax.fori_loop(0, CHR, finrow, 0)
            if with_xt:
                pltpu.sync_copy(pa_v, out_h26.at[pl.ds(rs, CHR)])
            pltpu.sync_copy(rv_v, out_tab.at[pl.ds(rs, CHR)])
            pltpu.sync_copy(rv_v, acc_sh.at[pl.ds(rs, CHR)])
        plsc.subcore_barrier()

        base = wid * EPW
        for b in range(NBUF):
            pltpu.async_copy(acc_sh.at[gidx_v.at[b]],
                             rows_v.at[pl.ds(b * CH, CH)], lsem.at[b])

        def gbody(g, carry):
            for b in range(NBUF):
                j = g * NBUF + b
                buf = rows_v.at[pl.ds(b * CH, CH)]
                pltpu.make_async_copy(acc_sh.at[gidx_v.at[j]], buf,
                                      lsem.at[b]).wait()
                pltpu.sync_copy(buf, out_hs.at[pl.ds(base + j * CH, CH)])
                jn = j + NBUF

                @pl.when(jn < NCHUNK)
                def _():
                    pltpu.async_copy(acc_sh.at[gidx_v.at[jn]], buf,
                                     lsem.at[b])
            return carry

        lax.fori_loop(0, NCHUNK // NBUF, gbody, 0)

    return functools.partial(
        pl.kernel, mesh=_mesh,
        out_type=outs,
        compiler_params=pltpu.CompilerParams(use_tc_tiling_on_sc=False),
        scratch_types=[
            pltpu.VMEM((NCHUNK2, CH), jnp.int32),
            pltpu.VMEM((NCHUNK, CH), jnp.int32),
            pltpu.VMEM((NBUF * CH, 32), jnp.float32),
            pltpu.SemaphoreType.DMA((NBUF,)),
            pltpu.SemaphoreType.DMA((NBUF,)),
            pltpu.VMEM((CHR, 32), jnp.float32),
            pltpu.VMEM((CHR, 32), jnp.float32),
            pltpu.VMEM((CHR, 32), jnp.float32),
            pltpu.VMEM((CHR, 32), jnp.float32),
            pltpu.VMEM_SHARED((NP, 32), jnp.float32),
        ],
    )(body)


_sc_scatfin_mid = _make_scatfin(False)
_sc_scatfin_step = _make_scatfin(True)


def _msg_body(hs_ref, attr_ref, w1_ref, b1_ref, t_ref, w2s_ref, bm_ref,
              htab_ref, root_ref, rbias_ref, cp_ref,
              out_ref, r_ref, invc_ref):
    @pl.when(pl.program_id(0) == 0)
    def _():
        r_ref[...] = (jnp.dot(htab_ref[...], root_ref[...],
                              preferred_element_type=jnp.float32)
                      + rbias_ref[...])
        cnt = cp_ref[0][:, 0:1] + cp_ref[1][:, 0:1]
        invc_ref[...] = jnp.broadcast_to(1.0 / jnp.maximum(cnt, 1.0),
                                         (NP, 32))

    hs = hs_ref[...]
    hsb = hs.astype(jnp.bfloat16)
    e = jnp.maximum(attr_ref[...] * w1_ref[...] + b1_ref[...], 0.0)
    e_tile = pltpu.repeat(e.astype(jnp.bfloat16), 32, axis=1)
    h_exp = jnp.dot(hsb, t_ref[...],
                    preferred_element_type=jnp.float32).astype(jnp.bfloat16)
    u = h_exp * e_tile
    out_ref[...] = (
        jnp.dot(u, w2s_ref[...], preferred_element_type=jnp.float32)
        + jnp.dot(hs, bm_ref[...], preferred_element_type=jnp.float32))


def _msg(hsrc, attrp, w1, b1, tmat, w2s, bmat, htab, root, rbias, cp):
    return pl.pallas_call(
        _msg_body,
        grid=(GRID_E,),
        in_specs=[
            pl.BlockSpec((BE, 32), lambda j: (j, 0)),
            pl.BlockSpec((BE, 1), lambda j: (j, 0)),
            pl.BlockSpec((1, 16), lambda j: (0, 0)),
            pl.BlockSpec((1, 16), lambda j: (0, 0)),
            pl.BlockSpec((32, 512), lambda j: (0, 0)),
            pl.BlockSpec((512, 32), lambda j: (0, 0)),
            pl.BlockSpec((32, 32), lambda j: (0, 0)),
            pl.BlockSpec((NP, 32), lambda j: (0, 0)),
            pl.BlockSpec((32, 32), lambda j: (0, 0)),
            pl.BlockSpec((1, 32), lambda j: (0, 0)),
            pl.BlockSpec((2, NP, 32), lambda j: (0, 0, 0)),
        ],
        out_specs=[
            pl.BlockSpec((BE, 32), lambda j: (j, 0)),
            pl.BlockSpec((NP, 32), lambda j: (0, 0)),
            pl.BlockSpec((NP, 32), lambda j: (0, 0)),
        ],
        out_shape=[
            jax.ShapeDtypeStruct((EP, 32), jnp.float32),
            jax.ShapeDtypeStruct((NP, 32), jnp.float32),
            jax.ShapeDtypeStruct((NP, 32), jnp.float32),
        ],
    )(hsrc, attrp, w1, b1, tmat, w2s, bmat, htab, root, rbias, cp)


def _prologue_body(bnd_ref, w1_ref, b1_ref, w2_ref, b2_ref, xt_ref, out_ref):
    h0 = jnp.maximum(bnd_ref[...] * w1_ref[...] + b1_ref[...], 0.0)
    h26 = jnp.maximum(
        jnp.dot(h0, w2_ref[...], preferred_element_type=jnp.float32)
        + b2_ref[...], 0.0)
    out_ref[...] = h26 + xt_ref[...]


def _prologue(bnd, fc1_w, fc1_b, fc2p, fc2bp, xt1):
    return pl.pallas_call(
        _prologue_body,
        out_shape=jax.ShapeDtypeStruct((NP, 32), jnp.float32),
    )(bnd, fc1_w, fc1_b, fc2p, fc2bp, xt1)


def _epi_body(h26a_ref, h26b_ref, p_ref, invc_ref, r_ref, fc3_ref, fc3b_ref,
              fc4_ref, fc4b_ref, y1_ref, y2_ref, y3_ref):
    h26c = jnp.maximum(
        (p_ref[0] + p_ref[1]) * invc_ref[...] + r_ref[...], 0.0)

    def head(h26, y_ref):
        z = jnp.maximum(
            jnp.dot(h26, fc3_ref[...], preferred_element_type=jnp.float32)
            + fc3b_ref[...], 0.0)
        y_ref[...] = (jnp.dot(z, fc4_ref[...],
                              preferred_element_type=jnp.float32)
                      + fc4b_ref[...])

    head(h26a_ref[...], y1_ref)
    head(h26b_ref[...], y2_ref)
    head(h26c, y3_ref)


def _epilogue(h26a, h26b, p, invc, r, fc3p, fc3b, fc4, fc4b):
    return pl.pallas_call(
        _epi_body,
        out_shape=[
            jax.ShapeDtypeStruct((NP, 1), jnp.float32),
            jax.ShapeDtypeStruct((NP, 1), jnp.float32),
            jax.ShapeDtypeStruct((NP, 1), jnp.float32),
        ],
    )(h26a, h26b, p, invc, r, fc3p, fc3b, fc4, fc4b)


def kernel(x, t, edge_index, edge_attr, y, fc1_w, fc1_b, fc2_w, fc2_b,
           nn1_w1, nn1_b1, nn1_w2, nn1_b2, conv1_root, conv1_bias,
           nn3_w1, nn3_b1, nn3_w2, nn3_b2, conv3_root, conv3_bias,
           fc3_w, fc3_b, fc4_w, fc4_b):
    f32 = jnp.float32
    pad_e = EP - E

    src = edge_index[0].astype(jnp.int32)
    dst = edge_index[1].astype(jnp.int32)
    src2d = jnp.concatenate([src, jnp.zeros((pad_e,), jnp.int32)]).reshape(-1, CH)
    dst2d = jnp.concatenate(
        [dst, jnp.full((pad_e,), N, jnp.int32)]).reshape(-1, CH)
    attrp = jnp.pad(edge_attr, ((0, pad_e), (0, 0)))

    w2s1 = nn1_w2.reshape(16, 32, 32).transpose(1, 0, 2).reshape(512, 32)
    bm1 = nn1_b2.reshape(32, 32)
    w2s3 = jnp.pad(nn3_w2.reshape(16, 32, 26),
                   ((0, 0), (0, 0), (0, 6))).transpose(1, 0, 2).reshape(512, 32)
    bm3 = jnp.pad(nn3_b2.reshape(32, 26), ((0, 0), (0, 6)))
    tmat = jnp.kron(jnp.eye(32, dtype=jnp.bfloat16),
                    jnp.ones((1, 16), jnp.bfloat16))
    w2s1 = w2s1.astype(jnp.bfloat16)
    w2s3 = w2s3.astype(jnp.bfloat16)
    w1e1 = nn1_w1
    b1e1 = nn1_b1.reshape(1, 16)
    w1e3 = nn3_w1
    b1e3 = nn3_b1.reshape(1, 16)
    root1 = conv1_root
    bias1 = conv1_bias.reshape(1, 32)
    root3 = jnp.pad(conv3_root, ((0, 0), (0, 6)))
    bias3 = jnp.pad(conv3_bias, (0, 6)).reshape(1, 32)
    fc2p = jnp.pad(fc2_w, ((0, 0), (0, 6)))
    fc2bp = jnp.pad(fc2_b, (0, 6)).reshape(1, 32)
    fc3p = jnp.pad(fc3_w, ((0, 6), (0, 0)))
    fc3b = fc3_b.reshape(1, 32)
    fc4b = fc4_b.reshape(1, 1)

    xp = jnp.pad(x, ((0, NP - N), (0, 0)))
    zeros26 = jnp.zeros((NP, 26), f32)

    def xt_for(ti):
        return jnp.concatenate(
            [zeros26, xp, xp, xp, jnp.broadcast_to(ti, (NP, 3))], axis=1)

    zeros_np = jnp.zeros((NP, 32), f32)
    ones_ch = jnp.ones((CH, 32), f32)

    bnd = jnp.pad(y[0].reshape(-1, 1), ((0, NP - N), (0, 0)))
    h32 = _prologue(bnd, fc1_w, fc1_b.reshape(1, 32), fc2p, fc2bp, xt_for(t[1]))

    h26s = []
    T = t.shape[0]
    hs1, cp = _sc_count_gather(h32, src2d, dst2d, ones_ch, zeros_np)
    for i in range(1, T):
        m1, r1, invc = _msg(hs1, attrp, w1e1, b1e1, tmat, w2s1, bm1,
                            h32, root1, bias1, cp)
        p1 = _sc_scatter(m1, dst2d, zeros_np)
        hs3, hl1 = _sc_gfin_mid(p1, invc, r1, invc, src2d)

        m3, r3, invc = _msg(hs3, attrp, w1e3, b1e3, tmat, w2s3, bm3,
                            hl1, root3, bias3, cp)
        p3 = _sc_scatter(m3, dst2d, zeros_np)
        if i + 1 < T:
            hs1, h32, h26 = _sc_gfin_step(p3, invc, r3, xt_for(t[i + 1]),
                                          src2d)
            h26s.append(h26)

    y1, y2, y3 = _epilogue(h26s[0], h26s[1], p3, invc, r3,
                           fc3p, fc3b, fc4_w, fc4b)
    return jnp.concatenate([y[0], y1[:N, 0], y2[:N, 0], y3[:N, 0]])


# speedup vs baseline: 1.0203x; 1.0042x over previous
import functools

import jax
import jax.numpy as jnp
from jax import lax
from jax.experimental import pallas as pl
from jax.experimental.pallas import tpu as pltpu
from jax.experimental.pallas import tpu_sc as plsc

N = 10000
E = 160000
NP = 10016
EP = 163840
NWORK = 32
EPW = EP // NWORK
CH = 128
NCHUNK = EPW // CH
ROWS_PER_SUB = NP // 16
CHR = ROWS_PER_SUB // 2
BE = 2048
GRID_E = EP // BE

_mesh = plsc.VectorSubcoreMesh(core_axis_name="c", subcore_axis_name="s")


NBUF = 4


@functools.partial(
    pl.kernel, mesh=_mesh,
    out_type=jax.ShapeDtypeStruct((EP, 32), jnp.float32),
    compiler_params=pltpu.CompilerParams(use_tc_tiling_on_sc=False),
    scratch_types=[
        pltpu.VMEM((NCHUNK, CH), jnp.int32),
        pltpu.VMEM((NBUF * CH, 32), jnp.float32),
        pltpu.SemaphoreType.DMA((NBUF,)),
    ],
)
def _sc_gather(table_hbm, src2d_hbm, out_hbm, idx_v, rows_v, gsem):
    c = lax.axis_index("c")
    s = lax.axis_index("s")
    wid = c * 16 + s
    pltpu.sync_copy(src2d_hbm.at[pl.ds(wid * NCHUNK, NCHUNK)], idx_v)
    base = wid * EPW
    for b in range(NBUF):
        pltpu.async_copy(table_hbm.at[idx_v.at[b]],
                         rows_v.at[pl.ds(b * CH, CH)], gsem.at[b])

    def body(g, carry):
        for b in range(NBUF):
            j = g * NBUF + b
            buf = rows_v.at[pl.ds(b * CH, CH)]
            pltpu.make_async_copy(table_hbm.at[idx_v.at[j]], buf,
                                  gsem.at[b]).wait()
            pltpu.sync_copy(buf, out_hbm.at[pl.ds(base + j * CH, CH)])
            jn = j + NBUF

            @pl.when(jn < NCHUNK)
            def _():
                pltpu.async_copy(table_hbm.at[idx_v.at[jn]], buf, gsem.at[b])
        return carry

    lax.fori_loop(0, NCHUNK // NBUF, body, 0)


@functools.partial(
    pl.kernel, mesh=_mesh,
    out_type=jax.ShapeDtypeStruct((2, NP, 32), jnp.float32),
    compiler_params=pltpu.CompilerParams(use_tc_tiling_on_sc=False),
    scratch_types=[
        pltpu.VMEM((NCHUNK, CH), jnp.int32),
        pltpu.VMEM((NBUF * CH, 32), jnp.float32),
        pltpu.SemaphoreType.DMA((NBUF,)),
        pltpu.SemaphoreType.DMA((NBUF,)),
        pltpu.VMEM_SHARED((NP, 32), jnp.float32),
    ],
)
def _sc_scatter(msg_hbm, dst2d_hbm, zeros_hbm, out_hbm, idx_v, rows_v, lsem,
                ssem, acc_sh):
    c = lax.axis_index("c")
    s = lax.axis_index("s")
    wid = c * 16 + s
    pltpu.sync_copy(zeros_hbm.at[pl.ds(s * ROWS_PER_SUB, ROWS_PER_SUB)],
                    acc_sh.at[pl.ds(s * ROWS_PER_SUB, ROWS_PER_SUB)])
    pltpu.sync_copy(dst2d_hbm.at[pl.ds(wid * NCHUNK, NCHUNK)], idx_v)
    plsc.subcore_barrier()
    base = wid * EPW
    for b in range(NBUF):
        pltpu.async_copy(msg_hbm.at[pl.ds(base + b * CH, CH)],
                         rows_v.at[pl.ds(b * CH, CH)], lsem.at[b])

    def body(g, carry):
        for b in range(NBUF):
            j = g * NBUF + b
            buf = rows_v.at[pl.ds(b * CH, CH)]
            pltpu.make_async_copy(msg_hbm.at[pl.ds(base + j * CH, CH)], buf,
                                  lsem.at[b]).wait()
            pltpu.async_copy(buf, acc_sh.at[idx_v.at[j]], ssem.at[b],
                             add=True)
            jn = j + NBUF

            @pl.when(jn < NCHUNK)
            def _():
                pltpu.make_async_copy(buf, acc_sh.at[idx_v.at[j]],
                                      ssem.at[b]).wait()
                pltpu.async_copy(msg_hbm.at[pl.ds(base + jn * CH, CH)], buf,
                                 lsem.at[b])
        return carry

    lax.fori_loop(0, NCHUNK // NBUF, body, 0)
    for b in range(NBUF):
        pltpu.make_async_copy(rows_v.at[pl.ds(b * CH, CH)],
                              acc_sh.at[idx_v.at[NCHUNK - NBUF + b]],
                              ssem.at[b]).wait()
    plsc.subcore_barrier()
    pltpu.sync_copy(acc_sh.at[pl.ds(s * ROWS_PER_SUB, ROWS_PER_SUB)],
                    out_hbm.at[c].at[pl.ds(s * ROWS_PER_SUB, ROWS_PER_SUB)])


@functools.partial(
    pl.kernel, mesh=_mesh,
    out_type=jax.ShapeDtypeStruct((2, NP, 32), jnp.float32),
    compiler_params=pltpu.CompilerParams(use_tc_tiling_on_sc=False),
    scratch_types=[
        pltpu.VMEM((NCHUNK, CH), jnp.int32),
        pltpu.VMEM((CH, 32), jnp.float32),
        pltpu.SemaphoreType.DMA((NBUF,)),
        pltpu.VMEM_SHARED((NP, 32), jnp.float32),
    ],
)
def _sc_count(ones_hbm, dst2d_hbm, zeros_hbm, out_hbm, idx_v, rows_v, csem,
              acc_sh):
    c = lax.axis_index("c")
    s = lax.axis_index("s")
    wid = c * 16 + s
    pltpu.sync_copy(zeros_hbm.at[pl.ds(s * ROWS_PER_SUB, ROWS_PER_SUB)],
                    acc_sh.at[pl.ds(s * ROWS_PER_SUB, ROWS_PER_SUB)])
    pltpu.sync_copy(dst2d_hbm.at[pl.ds(wid * NCHUNK, NCHUNK)], idx_v)
    pltpu.sync_copy(ones_hbm, rows_v)
    plsc.subcore_barrier()
    for b in range(NBUF):
        pltpu.async_copy(rows_v, acc_sh.at[idx_v.at[b]], csem.at[b], add=True)

    def body(g, carry):
        for b in range(NBUF):
            j = g * NBUF + b
            pltpu.make_async_copy(rows_v, acc_sh.at[idx_v.at[j]],
                                  csem.at[b]).wait()
            pltpu.async_copy(rows_v, acc_sh.at[idx_v.at[j + NBUF]],
                             csem.at[b], add=True)
        return carry

    lax.fori_loop(0, NCHUNK // NBUF - 1, body, 0)
    for b in range(NBUF):
        pltpu.make_async_copy(rows_v, acc_sh.at[idx_v.at[NCHUNK - NBUF + b]],
                              csem.at[b]).wait()
    plsc.subcore_barrier()
    pltpu.sync_copy(acc_sh.at[pl.ds(s * ROWS_PER_SUB, ROWS_PER_SUB)],
                    out_hbm.at[c].at[pl.ds(s * ROWS_PER_SUB, ROWS_PER_SUB)])


def _make_gfin(with_xt):
    outs = [
        jax.ShapeDtypeStruct((EP, 32), jnp.float32),
        jax.ShapeDtypeStruct((NP, 32), jnp.float32),
    ]
    if with_xt:
        outs.append(jax.ShapeDtypeStruct((NP, 32), jnp.float32))

    def body(p_hbm, invc_hbm, r_hbm, xt_hbm, src2d_hbm, *refs):
        if with_xt:
            out_hs, out_tab, out_h26 = refs[0], refs[1], refs[2]
            scr = refs[3:]
        else:
            out_hs, out_tab = refs[0], refs[1]
            out_h26 = None
            scr = refs[2:]
        idx_v, rows_v, gsem, pa_v, pb_v, iv_v, rv_v, xv_v, tab_sh = scr
        c = lax.axis_index("c")
        s = lax.axis_index("s")
        wid = c * 16 + s
        pltpu.sync_copy(src2d_hbm.at[pl.ds(wid * NCHUNK, NCHUNK)], idx_v)
        for k in range(2):
            rs = s * ROWS_PER_SUB + k * CHR
            pltpu.sync_copy(p_hbm.at[0].at[pl.ds(rs, CHR)], pa_v)
            pltpu.sync_copy(p_hbm.at[1].at[pl.ds(rs, CHR)], pb_v)
            pltpu.sync_copy(invc_hbm.at[pl.ds(rs, CHR)], iv_v)
            pltpu.sync_copy(r_hbm.at[pl.ds(rs, CHR)], rv_v)
            if with_xt:
                pltpu.sync_copy(xt_hbm.at[pl.ds(rs, CHR)], xv_v)

            def finrow(i, carry):
                for half in range(2):
                    sl = pl.ds(16 * half, 16)
                    v = (pa_v[i, sl] + pb_v[i, sl]) * iv_v[i, sl] + rv_v[i, sl]
                    h26 = jnp.maximum(v, 0.0)
                    if with_xt:
                        pa_v[i, sl] = h26
                        pb_v[i, sl] = h26 + xv_v[i, sl]
                    else:
                        pb_v[i, sl] = h26
                return carry

            lax.fori_loop(0, CHR, finrow, 0)
            if with_xt:
                pltpu.sync_copy(pa_v, out_h26.at[pl.ds(rs, CHR)])
            pltpu.sync_copy(pb_v, out_tab.at[pl.ds(rs, CHR)])
            pltpu.sync_copy(pb_v, tab_sh.at[pl.ds(rs, CHR)])
        plsc.subcore_barrier()

        base = wid * EPW
        for b in range(NBUF):
            pltpu.async_copy(tab_sh.at[idx_v.at[b]],
                             rows_v.at[pl.ds(b * CH, CH)], gsem.at[b])

        def gbody(g, carry):
            for b in range(NBUF):
                j = g * NBUF + b
                buf = rows_v.at[pl.ds(b * CH, CH)]
                pltpu.make_async_copy(tab_sh.at[idx_v.at[j]], buf,
                                      gsem.at[b]).wait()
                pltpu.sync_copy(buf, out_hs.at[pl.ds(base + j * CH, CH)])
                jn = j + NBUF

                @pl.when(jn < NCHUNK)
                def _():
                    pltpu.async_copy(tab_sh.at[idx_v.at[jn]], buf, gsem.at[b])
            return carry

        lax.fori_loop(0, NCHUNK // NBUF, gbody, 0)

    return functools.partial(
        pl.kernel, mesh=_mesh,
        out_type=outs,
        compiler_params=pltpu.CompilerParams(use_tc_tiling_on_sc=False),
        scratch_types=[
            pltpu.VMEM((NCHUNK, CH), jnp.int32),
            pltpu.VMEM((NBUF * CH, 32), jnp.float32),
            pltpu.SemaphoreType.DMA((NBUF,)),
            pltpu.VMEM((CHR, 32), jnp.float32),
            pltpu.VMEM((CHR, 32), jnp.float32),
            pltpu.VMEM((CHR, 32), jnp.float32),
            pltpu.VMEM((CHR, 32), jnp.float32),
            pltpu.VMEM((CHR, 32), jnp.float32),
            pltpu.VMEM_SHARED((NP, 32), jnp.float32),
        ],
    )(body)


_sc_gfin_mid = _make_gfin(False)
_sc_gfin_step = _make_gfin(True)


def _msg_body(hs_ref, attr_ref, w1_ref, b1_ref, t_ref, w2s_ref, bm_ref,
              htab_ref, root_ref, rbias_ref, cp_ref,
              out_ref, r_ref, invc_ref):
    @pl.when(pl.program_id(0) == 0)
    def _():
        r_ref[...] = (jnp.dot(htab_ref[...], root_ref[...],
                              preferred_element_type=jnp.float32)
                      + rbias_ref[...])
        cnt = cp_ref[0][:, 0:1] + cp_ref[1][:, 0:1]
        invc_ref[...] = jnp.broadcast_to(1.0 / jnp.maximum(cnt, 1.0),
                                         (NP, 32))

    hs = hs_ref[...]
    hsb = hs.astype(jnp.bfloat16)
    e = jnp.maximum(attr_ref[...] * w1_ref[...] + b1_ref[...], 0.0)
    e_tile = pltpu.repeat(e.astype(jnp.bfloat16), 32, axis=1)
    h_exp = jnp.dot(hsb, t_ref[...],
                    preferred_element_type=jnp.float32).astype(jnp.bfloat16)
    u = h_exp * e_tile
    out_ref[...] = (
        jnp.dot(u, w2s_ref[...], preferred_element_type=jnp.float32)
        + jnp.dot(hs, bm_ref[...], preferred_element_type=jnp.float32))


def _msg(hsrc, attrp, w1, b1, tmat, w2s, bmat, htab, root, rbias, cp):
    return pl.pallas_call(
        _msg_body,
        grid=(GRID_E,),
        in_specs=[
            pl.BlockSpec((BE, 32), lambda j: (j, 0)),
            pl.BlockSpec((BE, 1), lambda j: (j, 0)),
            pl.BlockSpec((1, 16), lambda j: (0, 0)),
            pl.BlockSpec((1, 16), lambda j: (0, 0)),
            pl.BlockSpec((32, 512), lambda j: (0, 0)),
            pl.BlockSpec((512, 32), lambda j: (0, 0)),
            pl.BlockSpec((32, 32), lambda j: (0, 0)),
            pl.BlockSpec((NP, 32), lambda j: (0, 0)),
            pl.BlockSpec((32, 32), lambda j: (0, 0)),
            pl.BlockSpec((1, 32), lambda j: (0, 0)),
            pl.BlockSpec((2, NP, 32), lambda j: (0, 0, 0)),
        ],
        out_specs=[
            pl.BlockSpec((BE, 32), lambda j: (j, 0)),
            pl.BlockSpec((NP, 32), lambda j: (0, 0)),
            pl.BlockSpec((NP, 32), lambda j: (0, 0)),
        ],
        out_shape=[
            jax.ShapeDtypeStruct((EP, 32), jnp.float32),
            jax.ShapeDtypeStruct((NP, 32), jnp.float32),
            jax.ShapeDtypeStruct((NP, 32), jnp.float32),
        ],
    )(hsrc, attrp, w1, b1, tmat, w2s, bmat, htab, root, rbias, cp)


def _prologue_body(bnd_ref, w1_ref, b1_ref, w2_ref, b2_ref, xt_ref, out_ref):
    h0 = jnp.maximum(bnd_ref[...] * w1_ref[...] + b1_ref[...], 0.0)
    h26 = jnp.maximum(
        jnp.dot(h0, w2_ref[...], preferred_element_type=jnp.float32)
        + b2_ref[...], 0.0)
    out_ref[...] = h26 + xt_ref[...]


def _prologue(bnd, fc1_w, fc1_b, fc2p, fc2bp, xt1):
    return pl.pallas_call(
        _prologue_body,
        out_shape=jax.ShapeDtypeStruct((NP, 32), jnp.float32),
    )(bnd, fc1_w, fc1_b, fc2p, fc2bp, xt1)


def _epi_body(h26a_ref, h26b_ref, p_ref, invc_ref, r_ref, fc3_ref, fc3b_ref,
              fc4_ref, fc4b_ref, y1_ref, y2_ref, y3_ref):
    h26c = jnp.maximum(
        (p_ref[0] + p_ref[1]) * invc_ref[...] + r_ref[...], 0.0)

    def head(h26, y_ref):
        z = jnp.maximum(
            jnp.dot(h26, fc3_ref[...], preferred_element_type=jnp.float32)
            + fc3b_ref[...], 0.0)
        y_ref[...] = (jnp.dot(z, fc4_ref[...],
                              preferred_element_type=jnp.float32)
                      + fc4b_ref[...])

    head(h26a_ref[...], y1_ref)
    head(h26b_ref[...], y2_ref)
    head(h26c, y3_ref)


def _epilogue(h26a, h26b, p, invc, r, fc3p, fc3b, fc4, fc4b):
    return pl.pallas_call(
        _epi_body,
        out_shape=[
            jax.ShapeDtypeStruct((NP, 1), jnp.float32),
            jax.ShapeDtypeStruct((NP, 1), jnp.float32),
            jax.ShapeDtypeStruct((NP, 1), jnp.float32),
        ],
    )(h26a, h26b, p, invc, r, fc3p, fc3b, fc4, fc4b)


def kernel(x, t, edge_index, edge_attr, y, fc1_w, fc1_b, fc2_w, fc2_b,
           nn1_w1, nn1_b1, nn1_w2, nn1_b2, conv1_root, conv1_bias,
           nn3_w1, nn3_b1, nn3_w2, nn3_b2, conv3_root, conv3_bias,
           fc3_w, fc3_b, fc4_w, fc4_b):
    f32 = jnp.float32
    pad_e = EP - E

    src = edge_index[0].astype(jnp.int32)
    dst = edge_index[1].astype(jnp.int32)
    src2d = jnp.concatenate([src, jnp.zeros((pad_e,), jnp.int32)]).reshape(-1, CH)
    dst2d = jnp.concatenate(
        [dst, jnp.full((pad_e,), N, jnp.int32)]).reshape(-1, CH)
    attrp = jnp.pad(edge_attr, ((0, pad_e), (0, 0)))

    w2s1 = nn1_w2.reshape(16, 32, 32).transpose(1, 0, 2).reshape(512, 32)
    bm1 = nn1_b2.reshape(32, 32)
    w2s3 = jnp.pad(nn3_w2.reshape(16, 32, 26),
                   ((0, 0), (0, 0), (0, 6))).transpose(1, 0, 2).reshape(512, 32)
    bm3 = jnp.pad(nn3_b2.reshape(32, 26), ((0, 0), (0, 6)))
    tmat = jnp.kron(jnp.eye(32, dtype=jnp.bfloat16),
                    jnp.ones((1, 16), jnp.bfloat16))
    w2s1 = w2s1.astype(jnp.bfloat16)
    w2s3 = w2s3.astype(jnp.bfloat16)
    w1e1 = nn1_w1
    b1e1 = nn1_b1.reshape(1, 16)
    w1e3 = nn3_w1
    b1e3 = nn3_b1.reshape(1, 16)
    root1 = conv1_root
    bias1 = conv1_bias.reshape(1, 32)
    root3 = jnp.pad(conv3_root, ((0, 0), (0, 6)))
    bias3 = jnp.pad(conv3_bias, (0, 6)).reshape(1, 32)
    fc2p = jnp.pad(fc2_w, ((0, 0), (0, 6)))
    fc2bp = jnp.pad(fc2_b, (0, 6)).reshape(1, 32)
    fc3p = jnp.pad(fc3_w, ((0, 6), (0, 0)))
    fc3b = fc3_b.reshape(1, 32)
    fc4b = fc4_b.reshape(1, 1)

    xp = jnp.pad(x, ((0, NP - N), (0, 0)))
    zeros26 = jnp.zeros((NP, 26), f32)

    def xt_for(ti):
        return jnp.concatenate(
            [zeros26, xp, xp, xp, jnp.broadcast_to(ti, (NP, 3))], axis=1)

    zeros_np = jnp.zeros((NP, 32), f32)
    ones_ch = jnp.ones((CH, 32), f32)

    cp = _sc_count(ones_ch, dst2d, zeros_np)

    bnd = jnp.pad(y[0].reshape(-1, 1), ((0, NP - N), (0, 0)))
    h32 = _prologue(bnd, fc1_w, fc1_b.reshape(1, 32), fc2p, fc2bp, xt_for(t[1]))

    h26s = []
    T = t.shape[0]
    hs1 = _sc_gather(h32, src2d)
    for i in range(1, T):
        m1, r1, invc = _msg(hs1, attrp, w1e1, b1e1, tmat, w2s1, bm1,
                            h32, root1, bias1, cp)
        p1 = _sc_scatter(m1, dst2d, zeros_np)
        hs3, hl1 = _sc_gfin_mid(p1, invc, r1, invc, src2d)

        m3, r3, invc = _msg(hs3, attrp, w1e3, b1e3, tmat, w2s3, bm3,
                            hl1, root3, bias3, cp)
        p3 = _sc_scatter(m3, dst2d, zeros_np)
        if i + 1 < T:
            hs1, h32, h26 = _sc_gfin_step(p3, invc, r3, xt_for(t[i + 1]),
                                          src2d)
            h26s.append(h26)

    y1, y2, y3 = _epilogue(h26s[0], h26s[1], p3, invc, r3,
                           fc3p, fc3b, fc4_w, fc4b)
    return jnp.concatenate([y[0], y1[:N, 0], y2[:N, 0], y3[:N, 0]])
